# Initial kernel scaffold; baseline (speedup 1.0000x reference)
import jax
import jax.numpy as jnp
from jax.experimental import pallas as pl


def kernel(prefix, table):
    raise NotImplementedError("write your pallas kernel here")


# speedup vs baseline: 1.5905x; 1.5905x over previous
import functools

import jax
import jax.numpy as jnp
from jax import lax
from jax.experimental import pallas as pl
from jax.experimental.pallas import tpu as pltpu
from jax.experimental.pallas import tpu_sc as plsc

D = 14336
B_TOTAL = 4096
NC, NS = 2, 16
NW = NC * NS
B_PER_W = B_TOTAL // NW
CH = 8
NCHUNK = B_PER_W // CH

_mesh = plsc.VectorSubcoreMesh(core_axis_name="c", subcore_axis_name="s")


@functools.partial(
    pl.kernel,
    mesh=_mesh,
    out_type=jax.ShapeDtypeStruct((B_TOTAL, D), jnp.float32),
    scratch_types=[
        pltpu.VMEM((B_PER_W,), jnp.int32),
        pltpu.VMEM((CH, D), jnp.float32),
        pltpu.SemaphoreType.DMA,
    ],
)
def _gather(table_hbm, idx_hbm, out_hbm, idx_v, rows_v, sem):
    wid = lax.axis_index("s") * NC + lax.axis_index("c")
    base = wid * B_PER_W
    pltpu.sync_copy(idx_hbm.at[pl.ds(base, B_PER_W)], idx_v)
    for c in range(NCHUNK):
        pltpu.async_copy(
            table_hbm.at[idx_v.at[pl.ds(c * CH, CH)]], rows_v, sem
        ).wait()
        pltpu.sync_copy(rows_v, out_hbm.at[pl.ds(base + c * CH, CH)])


def kernel(prefix, table):
    idx = prefix.reshape(-1).astype(jnp.int32)
    out = _gather(table, idx)
    return out.reshape(prefix.shape[0], prefix.shape[1], D)


# speedup vs baseline: 1.6427x; 1.0328x over previous
import functools

import jax
import jax.numpy as jnp
from jax import lax
from jax.experimental import pallas as pl
from jax.experimental.pallas import tpu as pltpu
from jax.experimental.pallas import tpu_sc as plsc

D = 14336
B_TOTAL = 4096
NC, NS = 2, 16
NW = NC * NS
B_PER_W = B_TOTAL // NW
CH = 4
NCHUNK = B_PER_W // CH

_mesh = plsc.VectorSubcoreMesh(core_axis_name="c", subcore_axis_name="s")


@functools.partial(
    pl.kernel,
    mesh=_mesh,
    out_type=jax.ShapeDtypeStruct((B_TOTAL, D), jnp.float32),
    scratch_types=[
        pltpu.VMEM((NCHUNK, CH), jnp.int32),
        pltpu.VMEM((CH, D), jnp.float32),
        pltpu.VMEM((CH, D), jnp.float32),
        pltpu.SemaphoreType.DMA,
        pltpu.SemaphoreType.DMA,
    ],
)
def _gather(table_hbm, idx_hbm, out_hbm, idx_v, rows0, rows1, sem_g, sem_o):
    wid = lax.axis_index("s") * NC + lax.axis_index("c")
    base = wid * B_PER_W
    pltpu.sync_copy(idx_hbm.at[wid], idx_v)
    bufs = (rows0, rows1)

    def start_gather(c):
        return pltpu.async_copy(table_hbm.at[idx_v.at[c]], bufs[c % 2], sem_g)

    def start_out(c):
        return pltpu.async_copy(
            bufs[c % 2], out_hbm.at[pl.ds(base + c * CH, CH)], sem_o
        )

    g = start_gather(0)
    o_prev = None
    for c in range(NCHUNK):
        g.wait()
        if o_prev is not None:
            o_prev.wait()
        if c + 1 < NCHUNK:
            g = start_gather(c + 1)
        o_prev = start_out(c)
    o_prev.wait()


def kernel(prefix, table):
    idx = prefix.reshape(NW, NCHUNK, CH).astype(jnp.int32)
    out = _gather(table, idx)
    return out.reshape(prefix.shape[0], prefix.shape[1], D)


# speedup vs baseline: 2.1517x; 1.3099x over previous
import functools

import jax
import jax.numpy as jnp
from jax import lax
from jax.experimental import pallas as pl
from jax.experimental.pallas import tpu as pltpu
from jax.experimental.pallas import tpu_sc as plsc

D = 14336
NROW = 128
B_TOTAL = 4096
NC, NS = 2, 16
NW = NC * NS
B_PER_W = B_TOTAL // NW
INFLIGHT = 16

_mesh = plsc.VectorSubcoreMesh(core_axis_name="c", subcore_axis_name="s")


@functools.partial(
    pl.kernel,
    mesh=_mesh,
    out_type=jax.ShapeDtypeStruct((B_TOTAL, D), jnp.float32),
    scratch_types=[
        pltpu.VMEM((B_PER_W,), jnp.int32),
        pltpu.VMEM_SHARED((NROW, D), jnp.float32),
        pltpu.SemaphoreType.DMA,
    ],
)
def _gather(table_hbm, idx_hbm, out_hbm, idx_v, table_sh, sem):
    cid = lax.axis_index("c")
    sid = lax.axis_index("s")
    wid = sid * NC + cid
    base = wid * B_PER_W
    rows_per_sub = NROW // NS
    pltpu.sync_copy(
        table_hbm.at[pl.ds(sid * rows_per_sub, rows_per_sub)],
        table_sh.at[pl.ds(sid * rows_per_sub, rows_per_sub)],
    )
    pltpu.sync_copy(idx_hbm.at[pl.ds(base, B_PER_W)], idx_v)
    plsc.subcore_barrier()
    copies = []
    for g in range(B_PER_W // 16):
        vec = idx_v[pl.ds(g * 16, 16)]
        for j in range(16):
            c = g * 16 + j
            if len(copies) >= INFLIGHT:
                copies.pop(0).wait()
            copies.append(
                pltpu.async_copy(
                    table_sh.at[vec[j]], out_hbm.at[base + c], sem
                )
            )
    for cp in copies:
        cp.wait()


def kernel(prefix, table):
    idx = prefix.reshape(-1).astype(jnp.int32)
    out = _gather(table, idx)
    return out.reshape(prefix.shape[0], prefix.shape[1], D)
